# Initial kernel scaffold; baseline (speedup 1.0000x reference)
import jax
import jax.numpy as jnp
from jax.experimental import pallas as pl


def kernel(input_ids, shared, soft_embedding):
    raise NotImplementedError("write your pallas kernel here")


# speedup vs baseline: 1.4394x; 1.4394x over previous
import functools

import jax
import jax.numpy as jnp
from jax import lax
from jax.experimental import pallas as pl
from jax.experimental.pallas import tpu as pltpu
from jax.experimental.pallas import tpu_sc as plsc

VOCAB = 32100
NSOFT = 100
HIDDEN = 1024
NTOK = 4 * 2048

NC = 2
NS = 16
L = 16
NW = NC * NS
TPW = NTOK // NW
C = 64
NCH = TPW // C


def _emb_body(ids_hbm, shared_hbm, soft_hbm, out_hbm, idx_v, hidx_v, rows_v, sem):
    wid = lax.axis_index("s") * NC + lax.axis_index("c")
    base = wid * TPW

    for ch in range(NCH):
        off = base + ch * C
        pltpu.sync_copy(ids_hbm.at[pl.ds(off, C)], idx_v)

        for j in range(C // L):
            v = idx_v[pl.ds(j * L, L)]
            hidx_v[pl.ds(j * L, L)] = jnp.where(v < VOCAB, v, 2)

        pltpu.async_copy(shared_hbm.at[hidx_v], rows_v, sem).wait()

        for g in range(C // L):
            gv = idx_v[pl.ds(g * L, L)]
            for k in range(L):
                sid = gv[k]

                @pl.when(sid >= VOCAB)
                def _(sid=sid, g=g, k=k):
                    pltpu.sync_copy(
                        soft_hbm.at[sid - VOCAB], rows_v.at[g * L + k]
                    )

        pltpu.sync_copy(rows_v, out_hbm.at[pl.ds(off, C)])


@functools.partial(
    pl.kernel,
    out_type=jax.ShapeDtypeStruct((NTOK, HIDDEN), jnp.float32),
    mesh=plsc.VectorSubcoreMesh(core_axis_name="c", subcore_axis_name="s"),
    scratch_types=[
        pltpu.VMEM((C,), jnp.int32),
        pltpu.VMEM((C,), jnp.int32),
        pltpu.VMEM((C, HIDDEN), jnp.float32),
        pltpu.SemaphoreType.DMA,
    ],
)
def _emb_lookup(ids_hbm, shared_hbm, soft_hbm, out_hbm, idx_v, hidx_v, rows_v, sem):
    _emb_body(ids_hbm, shared_hbm, soft_hbm, out_hbm, idx_v, hidx_v, rows_v, sem)


def kernel(input_ids, shared, soft_embedding):
    b, s = input_ids.shape
    ids = input_ids.reshape(-1).astype(jnp.int32)
    out = _emb_lookup(ids, shared, soft_embedding)
    return out.reshape(b, s, HIDDEN)


# speedup vs baseline: 1.5021x; 1.0436x over previous
import functools

import jax
import jax.numpy as jnp
from jax import lax
from jax.experimental import pallas as pl
from jax.experimental.pallas import tpu as pltpu
from jax.experimental.pallas import tpu_sc as plsc

VOCAB = 32100
NSOFT = 100
HIDDEN = 1024
NTOK = 4 * 2048

NC = 2
NS = 16
L = 16
NW = NC * NS
TPW = NTOK // NW
C = 32
NCH = TPW // C
NBUF = 2


def _emb_body(ids_hbm, shared_hbm, soft_hbm, out_hbm, idx_all, hidx_all,
              rows0, rows1, gsem, wsem):
    wid = lax.axis_index("s") * NC + lax.axis_index("c")
    base = wid * TPW
    rows = (rows0, rows1)

    pltpu.sync_copy(ids_hbm.at[pl.ds(base, TPW)], idx_all)

    for ch in range(NCH):
        for j in range(C // L):
            v = idx_all[pl.ds(ch * C + j * L, L)]
            hidx_all[ch, pl.ds(j * L, L)] = jnp.where(v < VOCAB, v, 2)

    def gather(ch):
        return pltpu.async_copy(
            shared_hbm.at[hidx_all.at[ch]], rows[ch % NBUF], gsem
        )

    def writeback(ch):
        return pltpu.async_copy(
            rows[ch % NBUF], out_hbm.at[pl.ds(base + ch * C, C)], wsem
        )

    gd = gather(0)
    wd_prev = None
    for ch in range(NCH):
        gd.wait()

        for g in range(C // L):
            gv = idx_all[pl.ds(ch * C + g * L, L)]
            for k in range(L):
                sid = gv[k]

                @pl.when(sid >= VOCAB)
                def _(sid=sid, g=g, k=k, ch=ch):
                    pltpu.sync_copy(
                        soft_hbm.at[sid - VOCAB], rows[ch % NBUF].at[g * L + k]
                    )

        if wd_prev is not None:
            wd_prev.wait()
        if ch + 1 < NCH:
            gd = gather(ch + 1)
        wd_prev = writeback(ch)
    wd_prev.wait()


@functools.partial(
    pl.kernel,
    out_type=jax.ShapeDtypeStruct((NTOK, HIDDEN), jnp.float32),
    mesh=plsc.VectorSubcoreMesh(core_axis_name="c", subcore_axis_name="s"),
    scratch_types=[
        pltpu.VMEM((TPW,), jnp.int32),
        pltpu.VMEM((NCH, C), jnp.int32),
        pltpu.VMEM((C, HIDDEN), jnp.float32),
        pltpu.VMEM((C, HIDDEN), jnp.float32),
        pltpu.SemaphoreType.DMA,
        pltpu.SemaphoreType.DMA,
    ],
)
def _emb_lookup(ids_hbm, shared_hbm, soft_hbm, out_hbm, idx_all, hidx_all,
                rows0, rows1, gsem, wsem):
    _emb_body(ids_hbm, shared_hbm, soft_hbm, out_hbm, idx_all, hidx_all,
              rows0, rows1, gsem, wsem)


def kernel(input_ids, shared, soft_embedding):
    b, s = input_ids.shape
    ids = input_ids.reshape(-1).astype(jnp.int32)
    out = _emb_lookup(ids, shared, soft_embedding)
    return out.reshape(b, s, HIDDEN)


# speedup vs baseline: 1.7026x; 1.1335x over previous
import functools

import jax
import jax.numpy as jnp
from jax import lax
from jax.experimental import pallas as pl
from jax.experimental.pallas import tpu as pltpu
from jax.experimental.pallas import tpu_sc as plsc

VOCAB = 32100
NSOFT = 100
HIDDEN = 1024
NTOK = 4 * 2048

NC = 2
NS = 16
L = 16
NW = NC * NS
TPW = NTOK // NW
C = 32
NCH = TPW // C
NBUF = 3


def _emb_body(ids_hbm, shared_hbm, soft_hbm, out_hbm, idx_all, hidx_all,
              rows0, rows1, rows2, gsem, wsem):
    wid = lax.axis_index("s") * NC + lax.axis_index("c")
    base = wid * TPW
    rows = (rows0, rows1, rows2)

    pltpu.sync_copy(ids_hbm.at[pl.ds(base, TPW)], idx_all)

    for ch in range(NCH):
        for j in range(C // L):
            v = idx_all[pl.ds(ch * C + j * L, L)]
            hidx_all[ch, pl.ds(j * L, L)] = jnp.where(v < VOCAB, v, 2)

    def gather(ch):
        return pltpu.async_copy(
            shared_hbm.at[hidx_all.at[ch]], rows[ch % NBUF], gsem
        )

    def writeback(ch):
        return pltpu.async_copy(
            rows[ch % NBUF], out_hbm.at[pl.ds(base + ch * C, C)], wsem
        )

    gds = {0: gather(0), 1: gather(1)}
    wds = {}
    for ch in range(NCH):
        gds.pop(ch).wait()

        for g in range(C // L):
            gv = idx_all[pl.ds(ch * C + g * L, L)]
            for k in range(L):
                sid = gv[k]

                @pl.when(sid >= VOCAB)
                def _(sid=sid, g=g, k=k, ch=ch):
                    pltpu.sync_copy(
                        soft_hbm.at[sid - VOCAB], rows[ch % NBUF].at[g * L + k]
                    )

        if ch + 2 < NCH:
            if ch - 1 >= 0:
                wds.pop(ch - 1).wait()
            gds[ch + 2] = gather(ch + 2)
        wds[ch] = writeback(ch)
    for ch in sorted(wds):
        wds.pop(ch).wait()


@functools.partial(
    pl.kernel,
    out_type=jax.ShapeDtypeStruct((NTOK, HIDDEN), jnp.float32),
    mesh=plsc.VectorSubcoreMesh(core_axis_name="c", subcore_axis_name="s"),
    scratch_types=[
        pltpu.VMEM((TPW,), jnp.int32),
        pltpu.VMEM((NCH, C), jnp.int32),
        pltpu.VMEM((C, HIDDEN), jnp.float32),
        pltpu.VMEM((C, HIDDEN), jnp.float32),
        pltpu.VMEM((C, HIDDEN), jnp.float32),
        pltpu.SemaphoreType.DMA,
        pltpu.SemaphoreType.DMA,
    ],
)
def _emb_lookup(ids_hbm, shared_hbm, soft_hbm, out_hbm, idx_all, hidx_all,
                rows0, rows1, rows2, gsem, wsem):
    _emb_body(ids_hbm, shared_hbm, soft_hbm, out_hbm, idx_all, hidx_all,
              rows0, rows1, rows2, gsem, wsem)


def kernel(input_ids, shared, soft_embedding):
    b, s = input_ids.shape
    ids = input_ids.reshape(-1).astype(jnp.int32)
    out = _emb_lookup(ids, shared, soft_embedding)
    return out.reshape(b, s, HIDDEN)


# speedup vs baseline: 1.7056x; 1.0017x over previous
import functools

import jax
import jax.numpy as jnp
from jax import lax
from jax.experimental import pallas as pl
from jax.experimental.pallas import tpu as pltpu
from jax.experimental.pallas import tpu_sc as plsc

VOCAB = 32100
NSOFT = 100
HIDDEN = 1024
BATCH = 4
SEQ = 2048
NTOK = BATCH * SEQ

NC = 2
NS = 16
L = 16
NW = NC * NS
TPW = NTOK // NW
WPR = SEQ // TPW
C = 32
NCH = TPW // C
NBUF = 3


def _emb_body(ids_hbm, shared_hbm, soft_hbm, out_hbm, idx_all, hidx_all,
              rows0, rows1, rows2, gsem, wsem):
    wid = lax.axis_index("s") * NC + lax.axis_index("c")
    brow = wid // WPR
    col0 = (wid % WPR) * TPW
    rows = (rows0, rows1, rows2)

    pltpu.sync_copy(ids_hbm.at[brow, pl.ds(col0, TPW)], idx_all)

    for ch in range(NCH):
        for j in range(C // L):
            v = idx_all[pl.ds(ch * C + j * L, L)]
            hidx_all[ch, pl.ds(j * L, L)] = jnp.where(v < VOCAB, v, 2)

    def gather(ch):
        return pltpu.async_copy(
            shared_hbm.at[hidx_all.at[ch]], rows[ch % NBUF], gsem
        )

    def writeback(ch):
        return pltpu.async_copy(
            rows[ch % NBUF],
            out_hbm.at[brow, pl.ds(col0 + ch * C, C)],
            wsem,
        )

    gds = {0: gather(0), 1: gather(1)}
    wds = {}
    for ch in range(NCH):
        gds.pop(ch).wait()

        for g in range(C // L):
            gv = idx_all[pl.ds(ch * C + g * L, L)]
            for k in range(L):
                sid = gv[k]

                @pl.when(sid >= VOCAB)
                def _(sid=sid, g=g, k=k, ch=ch):
                    pltpu.sync_copy(
                        soft_hbm.at[sid - VOCAB], rows[ch % NBUF].at[g * L + k]
                    )

        if ch + 2 < NCH:
            if ch - 1 >= 0:
                wds.pop(ch - 1).wait()
            gds[ch + 2] = gather(ch + 2)
        wds[ch] = writeback(ch)
    for ch in sorted(wds):
        wds.pop(ch).wait()


@functools.partial(
    pl.kernel,
    out_type=jax.ShapeDtypeStruct((BATCH, SEQ, HIDDEN), jnp.float32),
    mesh=plsc.VectorSubcoreMesh(core_axis_name="c", subcore_axis_name="s"),
    scratch_types=[
        pltpu.VMEM((TPW,), jnp.int32),
        pltpu.VMEM((NCH, C), jnp.int32),
        pltpu.VMEM((C, HIDDEN), jnp.float32),
        pltpu.VMEM((C, HIDDEN), jnp.float32),
        pltpu.VMEM((C, HIDDEN), jnp.float32),
        pltpu.SemaphoreType.DMA,
        pltpu.SemaphoreType.DMA,
    ],
)
def _emb_lookup(ids_hbm, shared_hbm, soft_hbm, out_hbm, idx_all, hidx_all,
                rows0, rows1, rows2, gsem, wsem):
    _emb_body(ids_hbm, shared_hbm, soft_hbm, out_hbm, idx_all, hidx_all,
              rows0, rows1, rows2, gsem, wsem)


def kernel(input_ids, shared, soft_embedding):
    ids = input_ids if input_ids.dtype == jnp.int32 else input_ids.astype(jnp.int32)
    return _emb_lookup(ids, shared, soft_embedding)


# speedup vs baseline: 1.7090x; 1.0020x over previous
import functools

import jax
import jax.numpy as jnp
from jax import lax
from jax.experimental import pallas as pl
from jax.experimental.pallas import tpu as pltpu
from jax.experimental.pallas import tpu_sc as plsc

VOCAB = 32100
NSOFT = 100
HIDDEN = 1024
BATCH = 4
SEQ = 2048
NTOK = BATCH * SEQ

NC = 2
NS = 16
L = 16
NW = NC * NS
TPW = NTOK // NW
WPR = SEQ // TPW
C = 16
NCH = TPW // C
NBUF = 7
AHEAD = 4


def _emb_body(ids_hbm, shared_hbm, soft_hbm, out_hbm, idx_all, hidx_all,
              *rest):
    rows = rest[:NBUF]
    gsem, wsem = rest[NBUF], rest[NBUF + 1]
    wid = lax.axis_index("s") * NC + lax.axis_index("c")
    brow = wid // WPR
    col0 = (wid % WPR) * TPW

    pltpu.sync_copy(ids_hbm.at[brow, pl.ds(col0, TPW)], idx_all)

    def clamp(ch):
        for j in range(C // L):
            v = idx_all[pl.ds(ch * C + j * L, L)]
            hidx_all[ch, pl.ds(j * L, L)] = jnp.where(v < VOCAB, v, 2)

    def gather(ch):
        return pltpu.async_copy(
            shared_hbm.at[hidx_all.at[ch]], rows[ch % NBUF], gsem
        )

    def writeback(ch):
        return pltpu.async_copy(
            rows[ch % NBUF],
            out_hbm.at[brow, pl.ds(col0 + ch * C, C)],
            wsem,
        )

    gds = {}
    wds = {}
    for ch in range(AHEAD):
        clamp(ch)
        gds[ch] = gather(ch)
    for ch in range(NCH):
        gds.pop(ch).wait()

        for g in range(C // L):
            gv = idx_all[pl.ds(ch * C + g * L, L)]
            for k in range(L):
                sid = gv[k]

                @pl.when(sid >= VOCAB)
                def _(sid=sid, g=g, k=k, ch=ch):
                    pltpu.sync_copy(
                        soft_hbm.at[sid - VOCAB], rows[ch % NBUF].at[g * L + k]
                    )

        nxt = ch + AHEAD
        if nxt < NCH:
            if nxt - NBUF >= 0:
                wds.pop(nxt - NBUF).wait()
            clamp(nxt)
            gds[nxt] = gather(nxt)
        wds[ch] = writeback(ch)
    for ch in sorted(wds):
        wds.pop(ch).wait()


@functools.partial(
    pl.kernel,
    out_type=jax.ShapeDtypeStruct((BATCH, SEQ, HIDDEN), jnp.float32),
    mesh=plsc.VectorSubcoreMesh(core_axis_name="c", subcore_axis_name="s"),
    scratch_types=[
        pltpu.VMEM((TPW,), jnp.int32),
        pltpu.VMEM((NCH, C), jnp.int32),
        *[pltpu.VMEM((C, HIDDEN), jnp.float32) for _ in range(NBUF)],
        pltpu.SemaphoreType.DMA,
        pltpu.SemaphoreType.DMA,
    ],
)
def _emb_lookup(ids_hbm, shared_hbm, soft_hbm, out_hbm, idx_all, hidx_all,
                *rest):
    _emb_body(ids_hbm, shared_hbm, soft_hbm, out_hbm, idx_all, hidx_all,
              *rest)


def kernel(input_ids, shared, soft_embedding):
    ids = input_ids if input_ids.dtype == jnp.int32 else input_ids.astype(jnp.int32)
    return _emb_lookup(ids, shared, soft_embedding)


# speedup vs baseline: 2.0619x; 1.2065x over previous
import functools

import jax
import jax.numpy as jnp
from jax import lax
from jax.experimental import pallas as pl
from jax.experimental.pallas import tpu as pltpu
from jax.experimental.pallas import tpu_sc as plsc

VOCAB = 32100
NSOFT = 100
HIDDEN = 1024
BATCH = 4
SEQ = 2048
NTOK = BATCH * SEQ

NC = 2
NS = 16
L = 16
NW = NC * NS
TPW = NTOK // NW
WPR = SEQ // TPW
C = 32
NCH = TPW // C
NG = TPW // L
NBUF = 3
AHEAD = 2


def _emb_body(ids_hbm, shared_hbm, soft_hbm, out_hbm, idx_all, id2d_v,
              hidx_all, patch_v, rows0, rows1, rows2, gsem, wsem):
    rows = (rows0, rows1, rows2)
    wid = lax.axis_index("s") * NC + lax.axis_index("c")
    brow = wid // WPR
    col0 = (wid % WPR) * TPW

    pltpu.sync_copy(ids_hbm.at[brow, pl.ds(col0, TPW)], idx_all)

    def clamp(ch):
        for j in range(C // L):
            v = idx_all[pl.ds(ch * C + j * L, L)]
            id2d_v[ch * (C // L) + j, pl.ds(0, L)] = v
            hidx_all[ch, pl.ds(j * L, L)] = jnp.where(v < VOCAB, v, 2)

    def gather(ch):
        return pltpu.async_copy(
            shared_hbm.at[hidx_all.at[ch]], rows[ch % NBUF], gsem
        )

    def writeback(ch):
        return pltpu.async_copy(
            rows[ch % NBUF],
            out_hbm.at[brow, pl.ds(col0 + ch * C, C)],
            wsem,
        )

    gds = {}
    wds = {}
    for ch in range(AHEAD):
        clamp(ch)
        gds[ch] = gather(ch)
    for ch in range(NCH):
        gds.pop(ch).wait()
        nxt = ch + AHEAD
        if nxt < NCH:
            if nxt - NBUF >= 0:
                wds.pop(nxt - NBUF).wait()
            clamp(nxt)
            gds[nxt] = gather(nxt)
        wds[ch] = writeback(ch)
    for ch in sorted(wds):
        wds.pop(ch).wait()

    def fix_group(g, carry):
        gv = id2d_v[g, pl.ds(0, L)]
        for k in range(L):
            sid = gv[k]

            @pl.when(sid >= VOCAB)
            def _(sid=sid, k=k):
                pltpu.sync_copy(soft_hbm.at[sid - VOCAB], patch_v)
                pltpu.sync_copy(
                    patch_v, out_hbm.at[brow, col0 + g * L + k]
                )

        return carry

    lax.fori_loop(0, NG, fix_group, 0)


@functools.partial(
    pl.kernel,
    out_type=jax.ShapeDtypeStruct((BATCH, SEQ, HIDDEN), jnp.float32),
    mesh=plsc.VectorSubcoreMesh(core_axis_name="c", subcore_axis_name="s"),
    scratch_types=[
        pltpu.VMEM((TPW,), jnp.int32),
        pltpu.VMEM((NG, L), jnp.int32),
        pltpu.VMEM((NCH, C), jnp.int32),
        pltpu.VMEM((HIDDEN,), jnp.float32),
        pltpu.VMEM((C, HIDDEN), jnp.float32),
        pltpu.VMEM((C, HIDDEN), jnp.float32),
        pltpu.VMEM((C, HIDDEN), jnp.float32),
        pltpu.SemaphoreType.DMA,
        pltpu.SemaphoreType.DMA,
    ],
)
def _emb_lookup(ids_hbm, shared_hbm, soft_hbm, out_hbm, idx_all, id2d_v,
                hidx_all, patch_v, rows0, rows1, rows2, gsem, wsem):
    _emb_body(ids_hbm, shared_hbm, soft_hbm, out_hbm, idx_all, id2d_v,
              hidx_all, patch_v, rows0, rows1, rows2, gsem, wsem)


def kernel(input_ids, shared, soft_embedding):
    ids = input_ids if input_ids.dtype == jnp.int32 else input_ids.astype(jnp.int32)
    return _emb_lookup(ids, shared, soft_embedding)


# speedup vs baseline: 2.0758x; 1.0067x over previous
import functools

import jax
import jax.numpy as jnp
from jax import lax
from jax.experimental import pallas as pl
from jax.experimental.pallas import tpu as pltpu
from jax.experimental.pallas import tpu_sc as plsc

VOCAB = 32100
NSOFT = 100
HIDDEN = 1024
BATCH = 4
SEQ = 2048
NTOK = BATCH * SEQ

NC = 2
NS = 16
L = 16
NW = NC * NS
TPW = NTOK // NW
WPR = SEQ // TPW
C = 32
NCH = TPW // C
NG = TPW // L
NBUF = 3
AHEAD = 2


def _emb_body(ids_hbm, shared_hbm, soft_hbm, out_hbm, idx_all, id2d_v,
              patch_v, rows0, rows1, rows2, gsem, wsem):
    rows = (rows0, rows1, rows2)
    wid = lax.axis_index("s") * NC + lax.axis_index("c")
    brow = wid // WPR
    col0 = (wid % WPR) * TPW

    pltpu.sync_copy(ids_hbm.at[brow, pl.ds(col0, TPW)], idx_all)

    def gather(ch):
        ds = []
        for j in range(C // L):
            v = idx_all[pl.ds(ch * C + j * L, L)]
            id2d_v[ch * (C // L) + j, pl.ds(0, L)] = v
            hv = jnp.where(v < VOCAB, v, 2)
            ds.append(
                pltpu.async_copy(
                    shared_hbm.at[hv],
                    rows[ch % NBUF].at[pl.ds(j * L, L)],
                    gsem,
                )
            )
        return ds

    def writeback(ch):
        return pltpu.async_copy(
            rows[ch % NBUF],
            out_hbm.at[brow, pl.ds(col0 + ch * C, C)],
            wsem,
        )

    gds = {}
    wds = {}
    for ch in range(AHEAD):
        gds[ch] = gather(ch)
    for ch in range(NCH):
        for d in gds.pop(ch):
            d.wait()
        nxt = ch + AHEAD
        if nxt < NCH:
            if nxt - NBUF >= 0:
                wds.pop(nxt - NBUF).wait()
            gds[nxt] = gather(nxt)
        wds[ch] = writeback(ch)
    for ch in sorted(wds):
        wds.pop(ch).wait()

    def fix_group(g, carry):
        gv = id2d_v[g, pl.ds(0, L)]
        for k in range(L):
            sid = gv[k]

            @pl.when(sid >= VOCAB)
            def _(sid=sid, k=k):
                pltpu.sync_copy(soft_hbm.at[sid - VOCAB], patch_v)
                pltpu.sync_copy(
                    patch_v, out_hbm.at[brow, col0 + g * L + k]
                )

        return carry

    lax.fori_loop(0, NG, fix_group, 0)


@functools.partial(
    pl.kernel,
    out_type=jax.ShapeDtypeStruct((BATCH, SEQ, HIDDEN), jnp.float32),
    mesh=plsc.VectorSubcoreMesh(core_axis_name="c", subcore_axis_name="s"),
    scratch_types=[
        pltpu.VMEM((TPW,), jnp.int32),
        pltpu.VMEM((NG, L), jnp.int32),
        pltpu.VMEM((HIDDEN,), jnp.float32),
        pltpu.VMEM((C, HIDDEN), jnp.float32),
        pltpu.VMEM((C, HIDDEN), jnp.float32),
        pltpu.VMEM((C, HIDDEN), jnp.float32),
        pltpu.SemaphoreType.DMA,
        pltpu.SemaphoreType.DMA,
    ],
)
def _emb_lookup(ids_hbm, shared_hbm, soft_hbm, out_hbm, idx_all, id2d_v,
                patch_v, rows0, rows1, rows2, gsem, wsem):
    _emb_body(ids_hbm, shared_hbm, soft_hbm, out_hbm, idx_all, id2d_v,
              patch_v, rows0, rows1, rows2, gsem, wsem)


def kernel(input_ids, shared, soft_embedding):
    ids = input_ids if input_ids.dtype == jnp.int32 else input_ids.astype(jnp.int32)
    return _emb_lookup(ids, shared, soft_embedding)
